# baseline (speedup 1.0000x reference)
import functools

import jax
import jax.numpy as jnp
from jax.experimental import pallas as pl
from jax.experimental.pallas import tpu as pltpu


def _linear_t_kernel(x_ref, w_ref, b_ref, o_ref):
    acc = jax.lax.dot_general(
        w_ref[...], x_ref[...],
        dimension_numbers=(((0,), (1,)), ((), ())),
        preferred_element_type=jnp.float32,
    ) + b_ref[...].T
    o_ref[...] = acc[: o_ref.shape[0], :]


@functools.partial(jax.jit, static_argnames=("output_size", "tm"))
def _forward(bow_vec, w_p, b_p, *, output_size, tm):
    B, F = bow_vec.shape
    F_pad, O_pad = w_p.shape

    out_t = pl.pallas_call(
        _linear_t_kernel,
        out_shape=jax.ShapeDtypeStruct((output_size, B), jnp.float32),
        grid=(B // tm,),
        in_specs=[
            pl.BlockSpec((tm, F_pad), lambda i: (i, 0)),
            pl.BlockSpec((F_pad, O_pad), lambda i: (0, 0)),
            pl.BlockSpec((1, O_pad), lambda i: (0, 0)),
        ],
        out_specs=pl.BlockSpec((output_size, tm), lambda i: (0, i)),
        compiler_params=pltpu.CompilerParams(
            dimension_semantics=("arbitrary",),
            vmem_limit_bytes=48 * 1024 * 1024,
        ),
    )(bow_vec, w_p, b_p)
    return out_t.T


def kernel(bow_vec, w_p, b_p):
    return _forward(bow_vec, w_p, b_p, output_size=100, tm=512)
